# Initial kernel scaffold; baseline (speedup 1.0000x reference)
import jax
import jax.numpy as jnp
from jax.experimental import pallas as pl


def kernel(x, edge_index, W1, att1, b1, W2, att2, b2):
    raise NotImplementedError("write your pallas kernel here")


# speedup vs baseline: 33.7811x; 33.7811x over previous
import functools

import jax
import jax.numpy as jnp
from jax import lax
from jax.experimental import pallas as pl
from jax.experimental.pallas import tpu as pltpu
from jax.experimental.pallas import tpu_sc as plsc

N = 10000
NPAD = 10112
E_RAW = 320000
E_TOT = E_RAW + N
NW = 32
CB = 128
NCHUNK = 81
EPAD = NW * CB * NCHUNK
RW = 144
ROWS_PER_TILE = NPAD // 16
NEG_SLOPE = 0.2
NBLK = NPAD // 128


def _splat(w, v):
  idx = jnp.full((16, 1), v, jnp.int32)
  dnums = lax.GatherDimensionNumbers(
      offset_dims=(), collapsed_slice_dims=(0,), start_index_map=(0,))
  return lax.gather(w, idx, dnums, slice_sizes=(1,),
                    mode=lax.GatherScatterMode.PROMISE_IN_BOUNDS)


def _make_edge_kernel(heads8):
  def body(hx_hbm, ad_hbm, src_hbm, dst_hbm, zeros_hbm, out_hbm,
           acc_sh, srcv, dstv, hxb, adb, msgb, sem_h, sem_a):
    c = lax.axis_index("c")
    s = lax.axis_index("s")
    wid = c * 16 + s
    r0 = s * ROWS_PER_TILE

    pltpu.sync_copy(zeros_hbm.at[pl.ds(r0, ROWS_PER_TILE)],
                    acc_sh.at[pl.ds(r0, ROWS_PER_TILE)])
    plsc.subcore_barrier()

    def chunk(j, _):
      pltpu.sync_copy(src_hbm.at[wid, j], srcv)
      pltpu.sync_copy(dst_hbm.at[wid, j], dstv)
      pltpu.async_copy(hx_hbm.at[srcv], hxb, sem_h).wait()
      pltpu.async_copy(ad_hbm.at[dstv], adb, sem_a).wait()

      def edge(e, _):
        a = adb[e, :] + hxb[e, pl.ds(128, 16)]
        w = jnp.exp(jnp.maximum(a, NEG_SLOPE * a))
        msgb[e, pl.ds(128, 16)] = w
        for v in range(8):
          hv = hxb[e, pl.ds(16 * v, 16)]
          wv = _splat(w, v if heads8 else 0)
          msgb[e, pl.ds(16 * v, 16)] = hv * wv
        return 0

      lax.fori_loop(0, CB, edge, 0)
      pltpu.sync_copy(msgb, acc_sh.at[dstv], add=True)
      return 0

    lax.fori_loop(0, NCHUNK, chunk, 0)
    plsc.subcore_barrier()
    pltpu.sync_copy(acc_sh.at[pl.ds(r0, ROWS_PER_TILE)],
                    out_hbm.at[c, pl.ds(r0, ROWS_PER_TILE)])

  return pl.kernel(
      body,
      out_type=jax.ShapeDtypeStruct((2, NPAD, RW), jnp.float32),
      mesh=plsc.VectorSubcoreMesh(
          core_axis_name="c", subcore_axis_name="s",
          num_cores=2, num_subcores=16),
      scratch_types=[
          pltpu.VMEM_SHARED((NPAD, RW), jnp.float32),
          pltpu.VMEM((CB,), jnp.int32),
          pltpu.VMEM((CB,), jnp.int32),
          pltpu.VMEM((CB, RW), jnp.float32),
          pltpu.VMEM((CB, 16), jnp.float32),
          pltpu.VMEM((CB, RW), jnp.float32),
          pltpu.SemaphoreType.DMA,
          pltpu.SemaphoreType.DMA,
      ],
      compiler_params=pltpu.CompilerParams(use_tc_tiling_on_sc=False),
  )


_edge_kernel_l1 = _make_edge_kernel(True)
_edge_kernel_l2 = _make_edge_kernel(False)


def _proj1_body(x_ref, w_ref, as_ref, ad_ref, hx_ref, ado_ref):
  h = jnp.dot(x_ref[...], w_ref[...], preferred_element_type=jnp.float32)
  hx_ref[:, pl.ds(0, 128)] = h
  hx_ref[:, pl.ds(128, 16)] = jnp.dot(h, as_ref[...],
                                      preferred_element_type=jnp.float32)
  ado_ref[...] = jnp.dot(h, ad_ref[...], preferred_element_type=jnp.float32)


def _proj1(xp, w1, as16, ad16):
  return pl.pallas_call(
      _proj1_body,
      grid=(NBLK,),
      in_specs=[
          pl.BlockSpec((128, 128), lambda i: (i, 0)),
          pl.BlockSpec((128, 128), lambda i: (0, 0)),
          pl.BlockSpec((128, 16), lambda i: (0, 0)),
          pl.BlockSpec((128, 16), lambda i: (0, 0)),
      ],
      out_specs=[
          pl.BlockSpec((128, RW), lambda i: (i, 0)),
          pl.BlockSpec((128, 16), lambda i: (i, 0)),
      ],
      out_shape=[
          jax.ShapeDtypeStruct((NPAD, RW), jnp.float32),
          jax.ShapeDtypeStruct((NPAD, 16), jnp.float32),
      ],
  )(xp, w1, as16, ad16)


def _mid_body(p0_ref, p1_ref, rep_ref, w2_ref, as_ref, ad_ref, b1_ref,
              hx_ref, ado_ref):
  num = p0_ref[:, pl.ds(0, 128)] + p1_ref[:, pl.ds(0, 128)]
  den = p0_ref[:, pl.ds(128, 16)] + p1_ref[:, pl.ds(128, 16)]
  den_rep = jnp.dot(den, rep_ref[...], preferred_element_type=jnp.float32)
  h1 = num / (den_rep + 1e-16) + b1_ref[...]
  h1 = jnp.where(h1 > 0, h1, jnp.exp(jnp.minimum(h1, 0.0)) - 1.0)
  h2 = jnp.dot(h1, w2_ref[...], preferred_element_type=jnp.float32)
  hx_ref[:, pl.ds(0, 128)] = h2
  hx_ref[:, pl.ds(128, 16)] = jnp.dot(h2, as_ref[...],
                                      preferred_element_type=jnp.float32)
  ado_ref[...] = jnp.dot(h2, ad_ref[...], preferred_element_type=jnp.float32)


def _mid(p0, p1, rep, w2, as2, ad2, b1r):
  return pl.pallas_call(
      _mid_body,
      grid=(NBLK,),
      in_specs=[
          pl.BlockSpec((128, RW), lambda i: (i, 0)),
          pl.BlockSpec((128, RW), lambda i: (i, 0)),
          pl.BlockSpec((16, 128), lambda i: (0, 0)),
          pl.BlockSpec((128, 128), lambda i: (0, 0)),
          pl.BlockSpec((128, 16), lambda i: (0, 0)),
          pl.BlockSpec((128, 16), lambda i: (0, 0)),
          pl.BlockSpec((1, 128), lambda i: (0, 0)),
      ],
      out_specs=[
          pl.BlockSpec((128, RW), lambda i: (i, 0)),
          pl.BlockSpec((128, 16), lambda i: (i, 0)),
      ],
      out_shape=[
          jax.ShapeDtypeStruct((NPAD, RW), jnp.float32),
          jax.ShapeDtypeStruct((NPAD, 16), jnp.float32),
      ],
  )(p0, p1, rep, w2, as2, ad2, b1r)


def _fin_body(q0_ref, q1_ref, b2_ref, out_ref):
  num = q0_ref[:, pl.ds(0, 128)] + q1_ref[:, pl.ds(0, 128)]
  den = q0_ref[:, pl.ds(128, 1)] + q1_ref[:, pl.ds(128, 1)]
  out_ref[...] = num / (den + 1e-16) + b2_ref[...]


def _fin(q0, q1, b2r):
  return pl.pallas_call(
      _fin_body,
      grid=(NBLK,),
      in_specs=[
          pl.BlockSpec((128, RW), lambda i: (i, 0)),
          pl.BlockSpec((128, RW), lambda i: (i, 0)),
          pl.BlockSpec((1, 128), lambda i: (0, 0)),
      ],
      out_specs=pl.BlockSpec((128, 128), lambda i: (i, 0)),
      out_shape=jax.ShapeDtypeStruct((NPAD, 128), jnp.float32),
  )(q0, q1, b2r)


@jax.jit
def kernel(x, edge_index, W1, att1, b1, W2, att2, b2):
  ei = edge_index.astype(jnp.int32)
  src0, dst0 = ei[0], ei[1]
  dst0 = jnp.where(src0 != dst0, dst0, N)
  loops = jnp.arange(N, dtype=jnp.int32)
  pad = EPAD - E_TOT
  src = jnp.concatenate([src0, loops, jnp.zeros((pad,), jnp.int32)])
  dst = jnp.concatenate([dst0, loops, jnp.full((pad,), N, jnp.int32)])
  src_r = src.reshape(NW, NCHUNK, CB)
  dst_r = dst.reshape(NW, NCHUNK, CB)

  xp = jnp.pad(x.astype(jnp.float32), ((0, NPAD - N), (0, 0)))
  zeros_hbm = jnp.zeros((NPAD, RW), jnp.float32)

  att_i = att1[0, :, :16]
  att_j = att1[0, :, 16:]
  eye8 = jnp.eye(8, dtype=jnp.float32)
  ad16 = jnp.pad((att_i[:, :, None] * eye8[:, None, :]).reshape(128, 8),
                 ((0, 0), (0, 8)))
  as16 = jnp.pad((att_j[:, :, None] * eye8[:, None, :]).reshape(128, 8),
                 ((0, 0), (0, 8)))
  ad2 = jnp.pad(att2[0, 0, :128][:, None], ((0, 0), (0, 15)))
  as2 = jnp.pad(att2[0, 0, 128:][:, None], ((0, 0), (0, 15)))
  rep = jnp.pad(jnp.kron(eye8, jnp.ones((1, 16), jnp.float32)),
                ((0, 8), (0, 0)))
  b1r = b1.reshape(1, 128)
  b2r = b2.reshape(1, 128)

  hx1, adst1 = _proj1(xp, W1, as16, ad16)
  part1 = _edge_kernel_l1(hx1, adst1, src_r, dst_r, zeros_hbm)
  hx2, adst2 = _mid(part1[0], part1[1], rep, W2, as2, ad2, b1r)
  part2 = _edge_kernel_l2(hx2, adst2, src_r, dst_r, zeros_hbm)
  out = _fin(part2[0], part2[1], b2r)
  return out[:N]


# speedup vs baseline: 41.8145x; 1.2378x over previous
import functools

import jax
import jax.numpy as jnp
from jax import lax
from jax.experimental import pallas as pl
from jax.experimental.pallas import tpu as pltpu
from jax.experimental.pallas import tpu_sc as plsc

N = 10000
NPAD = 10112
E_RAW = 320000
E_TOT = E_RAW + N
NW = 32
CB = 80
NCHUNK = 130
EPAD = NW * CB * (NCHUNK + 1)
RW = 144
ROWS_PER_TILE = NPAD // 16
NEG_SLOPE = 0.2
NBLK = NPAD // 128


def _splat(w, v):
  idx = jnp.full((16, 1), v, jnp.int32)
  dnums = lax.GatherDimensionNumbers(
      offset_dims=(), collapsed_slice_dims=(0,), start_index_map=(0,))
  return lax.gather(w, idx, dnums, slice_sizes=(1,),
                    mode=lax.GatherScatterMode.PROMISE_IN_BOUNDS)


def _make_edge_kernel(heads8):
  def body(hx_hbm, ad_hbm, src_hbm, dst_hbm, zeros_hbm, out_hbm,
           acc_sh, srcv0, srcv1, dstv0, dstv1, hxb0, hxb1, adb0, adb1,
           msgb, sem_h0, sem_h1, sem_a0, sem_a1):
    c = lax.axis_index("c")
    s = lax.axis_index("s")
    wid = c * 16 + s
    r0 = s * ROWS_PER_TILE
    srcv = (srcv0, srcv1)
    dstv = (dstv0, dstv1)
    hxb = (hxb0, hxb1)
    adb = (adb0, adb1)
    sem_h = (sem_h0, sem_h1)
    sem_a = (sem_a0, sem_a1)

    pltpu.sync_copy(zeros_hbm.at[pl.ds(r0, ROWS_PER_TILE)],
                    acc_sh.at[pl.ds(r0, ROWS_PER_TILE)])
    plsc.subcore_barrier()

    def fetch(j, b):
      pltpu.sync_copy(src_hbm.at[wid, j], srcv[b])
      pltpu.sync_copy(dst_hbm.at[wid, j], dstv[b])
      pltpu.async_copy(hx_hbm.at[srcv[b]], hxb[b], sem_h[b])
      pltpu.async_copy(ad_hbm.at[dstv[b]], adb[b], sem_a[b])

    def phase(j, b):
      fetch(j + 1, 1 - b)
      pltpu.make_async_copy(hx_hbm.at[srcv[b]], hxb[b], sem_h[b]).wait()
      pltpu.make_async_copy(ad_hbm.at[dstv[b]], adb[b], sem_a[b]).wait()

      @plsc.parallel_loop(0, CB, unroll=2)
      def _(e):
        a = adb[b][e, :] + hxb[b][e, pl.ds(128, 16)]
        w = jnp.exp(jnp.maximum(a, NEG_SLOPE * a))
        msgb[e, pl.ds(128, 16)] = w
        for v in range(8):
          hv = hxb[b][e, pl.ds(16 * v, 16)]
          wv = _splat(w, v if heads8 else 0)
          msgb[e, pl.ds(16 * v, 16)] = hv * wv

      pltpu.sync_copy(msgb, acc_sh.at[dstv[b]], add=True)

    fetch(0, 0)

    def chunk2(g, _):
      phase(2 * g, 0)
      phase(2 * g + 1, 1)
      return 0

    lax.fori_loop(0, NCHUNK // 2, chunk2, 0)
    pltpu.make_async_copy(hx_hbm.at[srcv[0]], hxb[0], sem_h[0]).wait()
    pltpu.make_async_copy(ad_hbm.at[dstv[0]], adb[0], sem_a[0]).wait()
    plsc.subcore_barrier()
    pltpu.sync_copy(acc_sh.at[pl.ds(r0, ROWS_PER_TILE)],
                    out_hbm.at[c, pl.ds(r0, ROWS_PER_TILE)])

  return pl.kernel(
      body,
      out_type=jax.ShapeDtypeStruct((2, NPAD, RW), jnp.float32),
      mesh=plsc.VectorSubcoreMesh(
          core_axis_name="c", subcore_axis_name="s",
          num_cores=2, num_subcores=16),
      scratch_types=[
          pltpu.VMEM_SHARED((NPAD, RW), jnp.float32),
          pltpu.VMEM((CB,), jnp.int32),
          pltpu.VMEM((CB,), jnp.int32),
          pltpu.VMEM((CB,), jnp.int32),
          pltpu.VMEM((CB,), jnp.int32),
          pltpu.VMEM((CB, RW), jnp.float32),
          pltpu.VMEM((CB, RW), jnp.float32),
          pltpu.VMEM((CB, 16), jnp.float32),
          pltpu.VMEM((CB, 16), jnp.float32),
          pltpu.VMEM((CB, RW), jnp.float32),
          pltpu.SemaphoreType.DMA,
          pltpu.SemaphoreType.DMA,
          pltpu.SemaphoreType.DMA,
          pltpu.SemaphoreType.DMA,
      ],
      compiler_params=pltpu.CompilerParams(use_tc_tiling_on_sc=False),
  )


_edge_kernel_l1 = _make_edge_kernel(True)
_edge_kernel_l2 = _make_edge_kernel(False)


def _proj1_body(x_ref, w_ref, as_ref, ad_ref, hx_ref, ado_ref):
  h = jnp.dot(x_ref[...], w_ref[...], preferred_element_type=jnp.float32)
  hx_ref[:, pl.ds(0, 128)] = h
  hx_ref[:, pl.ds(128, 16)] = jnp.dot(h, as_ref[...],
                                      preferred_element_type=jnp.float32)
  ado_ref[...] = jnp.dot(h, ad_ref[...], preferred_element_type=jnp.float32)


def _proj1(xp, w1, as16, ad16):
  return pl.pallas_call(
      _proj1_body,
      grid=(NBLK,),
      in_specs=[
          pl.BlockSpec((128, 128), lambda i: (i, 0)),
          pl.BlockSpec((128, 128), lambda i: (0, 0)),
          pl.BlockSpec((128, 16), lambda i: (0, 0)),
          pl.BlockSpec((128, 16), lambda i: (0, 0)),
      ],
      out_specs=[
          pl.BlockSpec((128, RW), lambda i: (i, 0)),
          pl.BlockSpec((128, 16), lambda i: (i, 0)),
      ],
      out_shape=[
          jax.ShapeDtypeStruct((NPAD, RW), jnp.float32),
          jax.ShapeDtypeStruct((NPAD, 16), jnp.float32),
      ],
  )(xp, w1, as16, ad16)


def _mid_body(p0_ref, p1_ref, rep_ref, w2_ref, as_ref, ad_ref, b1_ref,
              hx_ref, ado_ref):
  num = p0_ref[:, pl.ds(0, 128)] + p1_ref[:, pl.ds(0, 128)]
  den = p0_ref[:, pl.ds(128, 16)] + p1_ref[:, pl.ds(128, 16)]
  den_rep = jnp.dot(den, rep_ref[...], preferred_element_type=jnp.float32)
  h1 = num / (den_rep + 1e-16) + b1_ref[...]
  h1 = jnp.where(h1 > 0, h1, jnp.exp(jnp.minimum(h1, 0.0)) - 1.0)
  h2 = jnp.dot(h1, w2_ref[...], preferred_element_type=jnp.float32)
  hx_ref[:, pl.ds(0, 128)] = h2
  hx_ref[:, pl.ds(128, 16)] = jnp.dot(h2, as_ref[...],
                                      preferred_element_type=jnp.float32)
  ado_ref[...] = jnp.dot(h2, ad_ref[...], preferred_element_type=jnp.float32)


def _mid(p0, p1, rep, w2, as2, ad2, b1r):
  return pl.pallas_call(
      _mid_body,
      grid=(NBLK,),
      in_specs=[
          pl.BlockSpec((128, RW), lambda i: (i, 0)),
          pl.BlockSpec((128, RW), lambda i: (i, 0)),
          pl.BlockSpec((16, 128), lambda i: (0, 0)),
          pl.BlockSpec((128, 128), lambda i: (0, 0)),
          pl.BlockSpec((128, 16), lambda i: (0, 0)),
          pl.BlockSpec((128, 16), lambda i: (0, 0)),
          pl.BlockSpec((1, 128), lambda i: (0, 0)),
      ],
      out_specs=[
          pl.BlockSpec((128, RW), lambda i: (i, 0)),
          pl.BlockSpec((128, 16), lambda i: (i, 0)),
      ],
      out_shape=[
          jax.ShapeDtypeStruct((NPAD, RW), jnp.float32),
          jax.ShapeDtypeStruct((NPAD, 16), jnp.float32),
      ],
  )(p0, p1, rep, w2, as2, ad2, b1r)


def _fin_body(q0_ref, q1_ref, b2_ref, out_ref):
  num = q0_ref[:, pl.ds(0, 128)] + q1_ref[:, pl.ds(0, 128)]
  den = q0_ref[:, pl.ds(128, 1)] + q1_ref[:, pl.ds(128, 1)]
  out_ref[...] = num / (den + 1e-16) + b2_ref[...]


def _fin(q0, q1, b2r):
  return pl.pallas_call(
      _fin_body,
      grid=(NBLK,),
      in_specs=[
          pl.BlockSpec((128, RW), lambda i: (i, 0)),
          pl.BlockSpec((128, RW), lambda i: (i, 0)),
          pl.BlockSpec((1, 128), lambda i: (0, 0)),
      ],
      out_specs=pl.BlockSpec((128, 128), lambda i: (i, 0)),
      out_shape=jax.ShapeDtypeStruct((NPAD, 128), jnp.float32),
  )(q0, q1, b2r)


@jax.jit
def kernel(x, edge_index, W1, att1, b1, W2, att2, b2):
  ei = edge_index.astype(jnp.int32)
  src0, dst0 = ei[0], ei[1]
  dst0 = jnp.where(src0 != dst0, dst0, N)
  loops = jnp.arange(N, dtype=jnp.int32)
  pad = NW * CB * NCHUNK - E_TOT
  src = jnp.concatenate([src0, loops, jnp.zeros((pad,), jnp.int32)])
  dst = jnp.concatenate([dst0, loops, jnp.full((pad,), N, jnp.int32)])
  src_r = jnp.concatenate(
      [src.reshape(NW, NCHUNK, CB), jnp.zeros((NW, 1, CB), jnp.int32)], axis=1)
  dst_r = jnp.concatenate(
      [dst.reshape(NW, NCHUNK, CB), jnp.full((NW, 1, CB), N, jnp.int32)],
      axis=1)

  xp = jnp.pad(x.astype(jnp.float32), ((0, NPAD - N), (0, 0)))
  zeros_hbm = jnp.zeros((NPAD, RW), jnp.float32)

  att_i = att1[0, :, :16]
  att_j = att1[0, :, 16:]
  eye8 = jnp.eye(8, dtype=jnp.float32)
  ad16 = jnp.pad((att_i[:, :, None] * eye8[:, None, :]).reshape(128, 8),
                 ((0, 0), (0, 8)))
  as16 = jnp.pad((att_j[:, :, None] * eye8[:, None, :]).reshape(128, 8),
                 ((0, 0), (0, 8)))
  ad2 = jnp.pad(att2[0, 0, :128][:, None], ((0, 0), (0, 15)))
  as2 = jnp.pad(att2[0, 0, 128:][:, None], ((0, 0), (0, 15)))
  rep = jnp.pad(jnp.kron(eye8, jnp.ones((1, 16), jnp.float32)),
                ((0, 8), (0, 0)))
  b1r = b1.reshape(1, 128)
  b2r = b2.reshape(1, 128)

  hx1, adst1 = _proj1(xp, W1, as16, ad16)
  part1 = _edge_kernel_l1(hx1, adst1, src_r, dst_r, zeros_hbm)
  hx2, adst2 = _mid(part1[0], part1[1], rep, W2, as2, ad2, b1r)
  part2 = _edge_kernel_l2(hx2, adst2, src_r, dst_r, zeros_hbm)
  out = _fin(part2[0], part2[1], b2r)
  return out[:N]
